# Initial kernel scaffold; baseline (speedup 1.0000x reference)
import jax
import jax.numpy as jnp
from jax.experimental import pallas as pl


def kernel(nodes, adj, W0, b0, Wo, bo):
    raise NotImplementedError("write your pallas kernel here")


# speedup vs baseline: 1.5411x; 1.5411x over previous
import functools

import jax
import jax.numpy as jnp
from jax.experimental import pallas as pl


def _bxw_body(nodes_ref, w0_ref, b0_ref, out_ref):
    out_ref[0] = (
        jnp.dot(nodes_ref[0], w0_ref[...], preferred_element_type=jnp.float32)
        + b0_ref[0]
    )


def _gcn_body(bn, adj_ref, bxw_ref, nodes_ref, wo_ref, bo_ref, out_ref):
    i = pl.program_id(1)
    a = adj_ref[0]
    denom = jnp.sum(a, axis=1, keepdims=True) + 1.0
    axw = jnp.dot(
        a.astype(jnp.bfloat16),
        bxw_ref[0].astype(jnp.bfloat16),
        preferred_element_type=jnp.float32,
    )
    bxw_blk = bxw_ref[0, pl.ds(i * bn, bn), :]
    g = jnp.maximum((axw + bxw_blk) / denom, 0.0) + nodes_ref[0]
    out_ref[0] = (
        jnp.dot(g, wo_ref[...], preferred_element_type=jnp.float32) + bo_ref[0]
    )


def kernel(nodes, adj, W0, b0, Wo, bo):
    B, N, D = nodes.shape
    BN = 512

    bxw = pl.pallas_call(
        _bxw_body,
        grid=(B,),
        in_specs=[
            pl.BlockSpec((1, N, D), lambda b: (b, 0, 0)),
            pl.BlockSpec((D, D), lambda b: (0, 0)),
            pl.BlockSpec((1, D), lambda b: (0, 0)),
        ],
        out_specs=pl.BlockSpec((1, N, D), lambda b: (b, 0, 0)),
        out_shape=jax.ShapeDtypeStruct((B, N, D), jnp.float32),
    )(nodes, W0, b0.reshape(1, D))

    out = pl.pallas_call(
        functools.partial(_gcn_body, BN),
        grid=(B, N // BN),
        in_specs=[
            pl.BlockSpec((1, BN, N), lambda b, i: (b, i, 0)),
            pl.BlockSpec((1, N, D), lambda b, i: (b, 0, 0)),
            pl.BlockSpec((1, BN, D), lambda b, i: (b, i, 0)),
            pl.BlockSpec((D, D), lambda b, i: (0, 0)),
            pl.BlockSpec((1, D), lambda b, i: (0, 0)),
        ],
        out_specs=pl.BlockSpec((1, BN, D), lambda b, i: (b, i, 0)),
        out_shape=jax.ShapeDtypeStruct((B, N, D), jnp.float32),
    )(adj, bxw, nodes, Wo, bo.reshape(1, D))

    return out


# speedup vs baseline: 1.7191x; 1.1155x over previous
import jax
import jax.numpy as jnp
from jax.experimental import pallas as pl


def _gcn_body(adj_ref, nodes_all_ref, nodes_ref, w0_ref, b0_ref, wo_ref,
              bo_ref, out_ref):
    a = adj_ref[0]
    denom = jnp.sum(a, axis=1, keepdims=True) + 1.0
    h = jnp.dot(
        a.astype(jnp.bfloat16),
        nodes_all_ref[0].astype(jnp.bfloat16),
        preferred_element_type=jnp.float32,
    ) + nodes_ref[0]
    hw = jnp.dot(h, w0_ref[...], preferred_element_type=jnp.float32)
    g = jnp.maximum(hw / denom + b0_ref[0], 0.0) + nodes_ref[0]
    out_ref[0] = (
        jnp.dot(g, wo_ref[...], preferred_element_type=jnp.float32) + bo_ref[0]
    )


def kernel(nodes, adj, W0, b0, Wo, bo):
    B, N, D = nodes.shape
    BN = 512

    return pl.pallas_call(
        _gcn_body,
        grid=(B, N // BN),
        in_specs=[
            pl.BlockSpec((1, BN, N), lambda b, i: (b, i, 0)),
            pl.BlockSpec((1, N, D), lambda b, i: (b, 0, 0)),
            pl.BlockSpec((1, BN, D), lambda b, i: (b, i, 0)),
            pl.BlockSpec((D, D), lambda b, i: (0, 0)),
            pl.BlockSpec((1, D), lambda b, i: (0, 0)),
            pl.BlockSpec((D, D), lambda b, i: (0, 0)),
            pl.BlockSpec((1, D), lambda b, i: (0, 0)),
        ],
        out_specs=pl.BlockSpec((1, BN, D), lambda b, i: (b, i, 0)),
        out_shape=jax.ShapeDtypeStruct((B, N, D), jnp.float32),
    )(adj, nodes, nodes, W0, b0.reshape(1, D), Wo, bo.reshape(1, D))


# speedup vs baseline: 1.7225x; 1.0019x over previous
import jax
import jax.numpy as jnp
from jax.experimental import pallas as pl
from jax.experimental.pallas import tpu as pltpu


def _gcn_body(adj_ref, nodes_all_ref, nodes_ref, w0_ref, b0_ref, wo_ref,
              bo_ref, out_ref):
    a = adj_ref[0]
    denom = jnp.sum(a, axis=1, keepdims=True) + 1.0
    h = jnp.dot(
        a.astype(jnp.bfloat16),
        nodes_all_ref[0].astype(jnp.bfloat16),
        preferred_element_type=jnp.float32,
    ) + nodes_ref[0]
    hw = jnp.dot(h, w0_ref[...], preferred_element_type=jnp.float32)
    g = jnp.maximum(hw / denom + b0_ref[0], 0.0) + nodes_ref[0]
    out_ref[0] = (
        jnp.dot(g, wo_ref[...], preferred_element_type=jnp.float32) + bo_ref[0]
    )


def kernel(nodes, adj, W0, b0, Wo, bo):
    B, N, D = nodes.shape
    BN = 512

    return pl.pallas_call(
        _gcn_body,
        grid=(B, N // BN),
        in_specs=[
            pl.BlockSpec((1, BN, N), lambda b, i: (b, i, 0)),
            pl.BlockSpec((1, N, D), lambda b, i: (b, 0, 0)),
            pl.BlockSpec((1, BN, D), lambda b, i: (b, i, 0)),
            pl.BlockSpec((D, D), lambda b, i: (0, 0)),
            pl.BlockSpec((1, D), lambda b, i: (0, 0)),
            pl.BlockSpec((D, D), lambda b, i: (0, 0)),
            pl.BlockSpec((1, D), lambda b, i: (0, 0)),
        ],
        out_specs=pl.BlockSpec((1, BN, D), lambda b, i: (b, i, 0)),
        out_shape=jax.ShapeDtypeStruct((B, N, D), jnp.float32),
        compiler_params=pltpu.CompilerParams(
            dimension_semantics=("parallel", "parallel"),
        ),
    )(adj, nodes, nodes, W0, b0.reshape(1, D), Wo, bo.reshape(1, D))


# speedup vs baseline: 2.0831x; 1.2093x over previous
import jax
import jax.numpy as jnp
from jax.experimental import pallas as pl
from jax.experimental.pallas import tpu as pltpu


def _gcn_body(adj_ref, nodes_all_ref, nodes_ref, w0_ref, b0_ref, wo_ref,
              bo_ref, out_ref):
    a = adj_ref[0]
    denom = jnp.sum(a, axis=1, keepdims=True) + 1.0
    h = jnp.dot(
        a.astype(jnp.bfloat16),
        nodes_all_ref[0].astype(jnp.bfloat16),
        preferred_element_type=jnp.float32,
    ) + nodes_ref[0]
    hw = jnp.dot(h, w0_ref[...], preferred_element_type=jnp.float32)
    g = jnp.maximum(hw / denom + b0_ref[0], 0.0) + nodes_ref[0]
    out_ref[0] = (
        jnp.dot(g, wo_ref[...], preferred_element_type=jnp.float32) + bo_ref[0]
    )


def kernel(nodes, adj, W0, b0, Wo, bo):
    B, N, D = nodes.shape
    BN = 1024

    return pl.pallas_call(
        _gcn_body,
        grid=(B, N // BN),
        in_specs=[
            pl.BlockSpec((1, BN, N), lambda b, i: (b, i, 0)),
            pl.BlockSpec((1, N, D), lambda b, i: (b, 0, 0)),
            pl.BlockSpec((1, BN, D), lambda b, i: (b, i, 0)),
            pl.BlockSpec((D, D), lambda b, i: (0, 0)),
            pl.BlockSpec((1, D), lambda b, i: (0, 0)),
            pl.BlockSpec((D, D), lambda b, i: (0, 0)),
            pl.BlockSpec((1, D), lambda b, i: (0, 0)),
        ],
        out_specs=pl.BlockSpec((1, BN, D), lambda b, i: (b, i, 0)),
        out_shape=jax.ShapeDtypeStruct((B, N, D), jnp.float32),
        compiler_params=pltpu.CompilerParams(
            dimension_semantics=("parallel", "parallel"),
        ),
    )(adj, nodes, nodes, W0, b0.reshape(1, D), Wo, bo.reshape(1, D))
